# baseline (speedup 1.0000x reference)
import functools

import jax
import jax.numpy as jnp
from jax import lax
from jax.experimental import pallas as pl
from jax.experimental.pallas import tpu as pltpu
from jax.experimental.pallas import tpu_sc as plsc

_LANES = 16
_NUM_WORKERS = 32
_BLK = 128
_NBUF = 4


def _build(B, L, V, D):
    NB = B // _BLK
    D8 = D // 8
    assert NB == _NUM_WORKERS and L % _NBUF == 0

    mesh = plsc.VectorSubcoreMesh(core_axis_name="c", subcore_axis_name="s")

    @functools.partial(
        pl.kernel,
        out_type=jax.ShapeDtypeStruct((L, D8, NB, 8 * _BLK), jnp.float32),
        mesh=mesh,
        compiler_params=pltpu.CompilerParams(
            use_tc_tiling_on_sc=False, needs_layout_passes=False
        ),
        scratch_types=[
            pltpu.VMEM((L, _BLK), jnp.int32),
            pltpu.VMEM((L, D), jnp.float32),
            pltpu.VMEM((_NBUF, _BLK, D), jnp.float32),
            pltpu.VMEM((_NBUF, D8, 8 * _BLK), jnp.float32),
        ]
        + [pltpu.SemaphoreType.DMA] * (2 * _NBUF),
    )
    def k(xq_hbm, tok_hbm, pos_hbm, out_hbm, idx_t, pos_v, rows_v, tmp_v,
          *sems):
        gsems = sems[:_NBUF]
        ssems = sems[_NBUF:]
        cid = lax.axis_index("c")
        sid = lax.axis_index("s")
        wid = sid * 2 + cid
        b0 = wid * _BLK
        pltpu.sync_copy(xq_hbm.at[:, pl.ds(b0, _BLK)], idx_t)
        pltpu.sync_copy(pos_hbm, pos_v)
        iota = lax.iota(jnp.int32, _LANES)

        for par in range(_NBUF):
            pltpu.async_copy(
                tok_hbm.at[idx_t.at[par]], rows_v.at[par], gsems[par]
            )

        @pl.loop(0, L, step=_NBUF)
        def _(l):
            for par in range(_NBUF):
                ll = l + par
                rows = rows_v.at[par]
                tmp = tmp_v.at[par]
                gsem = gsems[par]
                ssem = ssems[par]

                pltpu.make_async_copy(
                    tok_hbm.at[idx_t.at[ll]], rows, gsem
                ).wait()

                @pl.when(ll >= _NBUF)
                def _():
                    pltpu.make_async_copy(
                        tmp, out_hbm.at[ll - _NBUF, :, wid], ssem
                    ).wait()

                lsplat = jnp.full((_LANES,), ll, jnp.int32)

                @plsc.parallel_loop(0, D)
                def _(i):
                    colv = (i & ~(_LANES - 1)) | ((i + iota) & (_LANES - 1))
                    prot = plsc.load_gather(pos_v, [lsplat, colv])
                    idx0 = colv >> 3
                    idx1b = (colv & 7) << 7
                    for g in range(_BLK // _LANES):
                        goff = g * _LANES + iota
                        rv = plsc.load_gather(rows, [goff, colv])
                        plsc.store_scatter(
                            tmp, [idx0, idx1b + goff], rv + prot
                        )

                pltpu.async_copy(tmp, out_hbm.at[ll, :, wid], ssem)

                @pl.when(ll + _NBUF < L)
                def _():
                    pltpu.async_copy(
                        tok_hbm.at[idx_t.at[ll + _NBUF]], rows, gsem
                    )

        for par in range(_NBUF):
            pltpu.make_async_copy(
                tmp_v.at[par], out_hbm.at[L - _NBUF + par, :, wid], ssems[par]
            ).wait()

    return k


def _tc_row_majorize(V, D, Vp):
    G = Vp // 2 // _BLK

    def body(a_ref, b_ref, o_ref):
        o_ref[:, 0:D] = a_ref[...].T
        o_ref[:, D:2 * D] = b_ref[...].T

    return pl.pallas_call(
        body,
        grid=(G,),
        in_specs=[
            pl.BlockSpec((D, _BLK), lambda i: (0, i)),
            pl.BlockSpec((D, _BLK), lambda i: (0, i + G)),
        ],
        out_specs=pl.BlockSpec((_BLK, 2 * D), lambda i: (i, 0)),
        out_shape=jax.ShapeDtypeStruct((Vp // 2, 2 * D), jnp.float32),
    )


def kernel(x, token_table, pos_table):
    B, L = x.shape
    V, D = token_table.shape
    Vp = -(-V // (2 * _BLK)) * (2 * _BLK)
    half = Vp // 2
    xq = jnp.transpose(x.astype(jnp.int32))
    xq = 2 * xq - jnp.where(xq >= half, Vp - 1, 0)
    tokT = jnp.transpose(token_table)
    tok2 = _tc_row_majorize(V, D, Vp)(tokT, tokT)
    tok_rm = tok2.reshape(Vp, D)
    out4 = _build(B, L, Vp, D)(xq, tok_rm, pos_table)
    out5 = out4.reshape(L, D // 8, B // _BLK, 8, _BLK)
    return out5.transpose(2, 4, 0, 1, 3).reshape(B, L, D)


# speedup vs baseline: 1.9064x; 1.9064x over previous
import functools

import jax
import jax.numpy as jnp
from jax import lax
from jax.experimental import pallas as pl
from jax.experimental.pallas import tpu as pltpu
from jax.experimental.pallas import tpu_sc as plsc

_LANES = 16
_NUM_WORKERS = 32
_BLK = 128
_NBUF = 4


def _build(B, L, V, D):
    NB = B // _BLK
    D8 = D // 8
    assert NB == _NUM_WORKERS and L % _NBUF == 0

    mesh = plsc.VectorSubcoreMesh(core_axis_name="c", subcore_axis_name="s")

    @functools.partial(
        pl.kernel,
        out_type=jax.ShapeDtypeStruct((L, D8, NB, 8 * _BLK), jnp.float32),
        mesh=mesh,
        compiler_params=pltpu.CompilerParams(
            use_tc_tiling_on_sc=False, needs_layout_passes=False
        ),
        scratch_types=[
            pltpu.VMEM((L, _BLK), jnp.int32),
            pltpu.VMEM((L, D), jnp.float32),
            pltpu.VMEM((_NBUF, _BLK, D), jnp.float32),
            pltpu.VMEM((_NBUF, D8, 8 * _BLK), jnp.float32),
        ]
        + [pltpu.SemaphoreType.DMA] * (2 * _NBUF),
    )
    def k(xq_hbm, tok_hbm, pos_hbm, out_hbm, idx_t, pos_v, rows_v, tmp_v,
          *sems):
        gsems = sems[:_NBUF]
        ssems = sems[_NBUF:]
        cid = lax.axis_index("c")
        sid = lax.axis_index("s")
        wid = sid * 2 + cid
        b0 = wid * _BLK
        pltpu.sync_copy(xq_hbm.at[:, pl.ds(b0, _BLK)], idx_t)
        pltpu.sync_copy(pos_hbm, pos_v)
        iota = lax.iota(jnp.int32, _LANES)

        for par in range(_NBUF):
            pltpu.async_copy(
                tok_hbm.at[idx_t.at[par]], rows_v.at[par], gsems[par]
            )

        @pl.loop(0, L, step=_NBUF)
        def _(l):
            for par in range(_NBUF):
                ll = l + par
                rows = rows_v.at[par]
                tmp = tmp_v.at[par]
                gsem = gsems[par]
                ssem = ssems[par]

                pltpu.make_async_copy(
                    tok_hbm.at[idx_t.at[ll]], rows, gsem
                ).wait()

                @pl.when(ll >= _NBUF)
                def _():
                    pltpu.make_async_copy(
                        tmp, out_hbm.at[ll - _NBUF, :, wid], ssem
                    ).wait()

                lsplat = jnp.full((_LANES,), ll, jnp.int32)

                @plsc.parallel_loop(0, D)
                def _(i):
                    colv = (i & ~(_LANES - 1)) | ((i + iota) & (_LANES - 1))
                    prot = plsc.load_gather(pos_v, [lsplat, colv])
                    idx0 = colv >> 3
                    idx1b = (colv & 7) << 7
                    for g in range(_BLK // _LANES):
                        goff = g * _LANES + iota
                        rv = plsc.load_gather(rows, [goff, colv])
                        plsc.store_scatter(
                            tmp, [idx0, idx1b + goff], rv + prot
                        )

                pltpu.async_copy(tmp, out_hbm.at[ll, :, wid], ssem)

                @pl.when(ll + _NBUF < L)
                def _():
                    pltpu.async_copy(
                        tok_hbm.at[idx_t.at[ll + _NBUF]], rows, gsem
                    )

        for par in range(_NBUF):
            pltpu.make_async_copy(
                tmp_v.at[par], out_hbm.at[L - _NBUF + par, :, wid], ssems[par]
            ).wait()

    return k


def _tc_row_majorize(V, D, Vp):
    G = 23
    W = Vp // 2 // G

    def body(a_ref, b_ref, o_ref):
        o_ref[:, 0:D] = a_ref[...].T
        o_ref[:, D:2 * D] = b_ref[...].T

    return pl.pallas_call(
        body,
        grid=(G,),
        in_specs=[
            pl.BlockSpec((D, W), lambda i: (0, i)),
            pl.BlockSpec((D, W), lambda i: (0, i + G)),
        ],
        out_specs=pl.BlockSpec((W, 2 * D), lambda i: (i, 0)),
        out_shape=jax.ShapeDtypeStruct((Vp // 2, 2 * D), jnp.float32),
    )


def kernel(x, token_table, pos_table):
    B, L = x.shape
    V, D = token_table.shape
    Vp = -(-V // (2 * _BLK)) * (2 * _BLK)
    half = Vp // 2
    xq = jnp.transpose(x.astype(jnp.int32))
    xq = 2 * xq - jnp.where(xq >= half, Vp - 1, 0)
    tokT = jnp.transpose(token_table)
    tok2 = _tc_row_majorize(V, D, Vp)(tokT, tokT)
    tok_rm = tok2.reshape(Vp, D)
    out4 = _build(B, L, Vp, D)(xq, tok_rm, pos_table)
    out5 = out4.reshape(L, D // 8, B // _BLK, 8, _BLK)
    return out5.transpose(2, 4, 0, 1, 3).reshape(B, L, D)
